# baseline (speedup 1.0000x reference)
import functools

import jax
import jax.numpy as jnp
from jax import lax
from jax.experimental import pallas as pl
from jax.experimental.pallas import tpu as pltpu
from jax.experimental.pallas import tpu_sc as plsc

N_EMB = 8192
D = 32
N_TOK = 8192
TOK_BLK = 1024
CODE_BLK = 2048
N_TOK_BLKS = N_TOK // TOK_BLK
CNT_BLK = N_EMB // N_TOK_BLKS

_NC, _NS = 2, 16
_NW = _NC * _NS
_BPW = N_TOK // _NW
_CHUNK = 128
_NCHUNK = _BPW // _CHUNK


def _argmin_body(ze_ref, emb_ref, z2_ref, e2_ref, idx_ref):
    z = ze_ref[...]
    z2 = z2_ref[...]
    half = N_EMB // 2
    n_chunks = half // CODE_BLK
    n_cols = CODE_BLK // 128

    def half_argmin(h):
        def step(c, carry):
            rv, rc = carry
            base = h * half + c * CODE_BLK
            e = emb_ref[pl.ds(base, CODE_BLK), :]
            e2 = e2_ref[0, pl.ds(base, CODE_BLK)]
            mm = lax.dot_general(z, e, (((1,), (1,)), ((), ())),
                                 preferred_element_type=jnp.float32)
            dist = (z2 + e2[None, :]) - 2.0 * mm
            d3 = dist.reshape(TOK_BLK, n_cols, 128)
            for j in range(n_cols):
                v = d3[:, j, :]
                upd = v < rv
                rv = jnp.where(upd, v, rv)
                rc = jnp.where(upd, c * n_cols + j, rc)
            return rv, rc

        rv0 = jnp.full((TOK_BLK, 128), jnp.inf, dtype=jnp.float32)
        rc0 = jnp.zeros((TOK_BLK, 128), dtype=jnp.int32)
        rv, rc = lax.fori_loop(0, n_chunks, step, (rv0, rc0))
        lane = lax.broadcasted_iota(jnp.int32, (TOK_BLK, 128), 1)
        fidx = (rc * 128 + lane) + h * half
        lv = jnp.min(rv, axis=1)
        li = jnp.min(jnp.where(rv == lv[:, None], fidx, N_EMB), axis=1)
        return lv, li

    bvl, bil = half_argmin(0)
    bvr, bir = half_argmin(1)
    bvl_q = bvl.astype(jnp.bfloat16).astype(jnp.float32)
    idx_ref[0, 0, :] = jnp.where(bvr < bvl_q, bir, bil)


_argmin_call = pl.pallas_call(
    _argmin_body,
    grid=(N_TOK_BLKS,),
    in_specs=[
        pl.BlockSpec((TOK_BLK, D), lambda i: (i, 0)),
        pl.BlockSpec((N_EMB, D), lambda i: (0, 0)),
        pl.BlockSpec((TOK_BLK, 1), lambda i: (i, 0)),
        pl.BlockSpec((1, N_EMB), lambda i: (0, 0)),
    ],
    out_specs=pl.BlockSpec((1, 1, TOK_BLK), lambda i: (i, 0, 0)),
    out_shape=jax.ShapeDtypeStruct((N_TOK_BLKS, 1, TOK_BLK), jnp.int32),
)


_DPAD = 128


@functools.cache
def _build_sc_gather():
    @functools.partial(
        pl.kernel,
        out_type=jax.ShapeDtypeStruct((N_TOK, _DPAD), jnp.float32),
        mesh=plsc.VectorSubcoreMesh(core_axis_name="c", subcore_axis_name="s"),
        scratch_types=[
            pltpu.VMEM((_NCHUNK, _CHUNK), jnp.int32),
            pltpu.VMEM((_BPW, _DPAD), jnp.float32),
            pltpu.SemaphoreType.DMA,
        ],
    )
    def _sc_gather(idx_hbm, emb_hbm, zq_hbm, idx_v, rows_v, sem):
        wid = lax.axis_index("s") * _NC + lax.axis_index("c")
        base = wid * _BPW
        pltpu.sync_copy(idx_hbm.at[wid], idx_v)
        copies = [
            pltpu.async_copy(emb_hbm.at[idx_v.at[j]],
                             rows_v.at[pl.ds(j * _CHUNK, _CHUNK)], sem)
            for j in range(_NCHUNK)
        ]
        for cp in copies:
            cp.wait()
        pltpu.sync_copy(rows_v, zq_hbm.at[pl.ds(base, _BPW)])

    return _sc_gather


def _finalize_body(ze_ref, zq_ref, idx_ref, st_ref, loss_ref, perp_ref,
                   sse_ref, ent_ref):
    i = pl.program_id(0)

    @pl.when(i == 0)
    def _():
        sse_ref[0] = 0.0
        ent_ref[0] = 0.0

    ze = ze_ref[...]
    zq = zq_ref[:, :D]
    diff = zq - ze
    st_ref[...] = ze + diff
    sse_ref[0] += jnp.sum(diff * diff)

    idx = idx_ref[...]
    jcol = i * CNT_BLK + lax.broadcasted_iota(jnp.int32, (1, CNT_BLK), 1)
    counts = jnp.sum((idx == jcol).astype(jnp.float32), axis=0)
    p = counts * (1.0 / N_TOK)
    ent_ref[0] += jnp.sum(p * jnp.log(p + 1e-10))

    @pl.when(i == N_TOK_BLKS - 1)
    def _():
        loss_ref[0, 0] = sse_ref[0] * (1.25 / (N_TOK * D))
        perp_ref[0, 0] = jnp.exp(-ent_ref[0])


_finalize_call = pl.pallas_call(
    _finalize_body,
    grid=(N_TOK_BLKS,),
    in_specs=[
        pl.BlockSpec((TOK_BLK, D), lambda i: (i, 0)),
        pl.BlockSpec((TOK_BLK, _DPAD), lambda i: (i, 0)),
        pl.BlockSpec((N_TOK, 1), lambda i: (0, 0)),
    ],
    out_specs=[
        pl.BlockSpec((TOK_BLK, D), lambda i: (i, 0)),
        pl.BlockSpec((1, 1), lambda i: (0, 0), memory_space=pltpu.SMEM),
        pl.BlockSpec((1, 1), lambda i: (0, 0), memory_space=pltpu.SMEM),
    ],
    out_shape=[
        jax.ShapeDtypeStruct((N_TOK, D), jnp.float32),
        jax.ShapeDtypeStruct((1, 1), jnp.float32),
        jax.ShapeDtypeStruct((1, 1), jnp.float32),
    ],
    scratch_shapes=[
        pltpu.SMEM((1,), jnp.float32),
        pltpu.SMEM((1,), jnp.float32),
    ],
)


def kernel(z_e, embedding):
    z_e_flat = z_e.reshape(N_TOK, D)
    z2 = jnp.sum(z_e_flat ** 2, axis=1, keepdims=True)
    e2 = jnp.sum(embedding ** 2, axis=1).reshape(1, N_EMB)
    idx3 = _argmin_call(z_e_flat, embedding, z2, e2)
    indices = idx3.reshape(N_TOK)
    emb_pad = jnp.pad(embedding, ((0, 0), (0, _DPAD - D)))
    zq_pad = _build_sc_gather()(indices.reshape(_NW, _NCHUNK, _CHUNK),
                                emb_pad)
    st, loss, perp = _finalize_call(z_e_flat, zq_pad,
                                    indices.reshape(N_TOK, 1))
    return st.reshape(z_e.shape), loss[0, 0], perp[0, 0]


# speedup vs baseline: 4.2071x; 4.2071x over previous
import functools

import jax
import jax.numpy as jnp
from jax import lax
from jax.experimental import pallas as pl
from jax.experimental.pallas import tpu as pltpu
from jax.experimental.pallas import tpu_sc as plsc

N_EMB = 8192
D = 32
N_TOK = 8192
TOK_BLK = 512
CODE_BLK = 2048
N_TOK_BLKS = N_TOK // TOK_BLK
CNT_BLK = N_EMB // N_TOK_BLKS

_NC, _NS = 2, 16
_NW = _NC * _NS
_BPW = N_TOK // _NW
_CHUNK = 128
_NCHUNK = _BPW // _CHUNK


def _argmin_body(ze_ref, emb_ref, z2_ref, e2_ref, idx_ref):
    z = ze_ref[...]
    z2 = z2_ref[...]

    def step(c, carry):
        bvl, bil, bvr, bir = carry
        e = emb_ref[pl.ds(c * CODE_BLK, CODE_BLK), :]
        e2 = e2_ref[0, pl.ds(c * CODE_BLK, CODE_BLK)]
        mm = lax.dot_general(z, e, (((1,), (1,)), ((), ())),
                             preferred_element_type=jnp.float32)
        dist = (z2 + e2[None, :]) - 2.0 * mm
        lv = jnp.min(dist, axis=1)
        ii = lax.broadcasted_iota(jnp.int32, dist.shape, 1)
        li = jnp.min(jnp.where(dist == lv[:, None], ii, N_EMB), axis=1)
        li = li + c * CODE_BLK
        is_left = c < (N_EMB // CODE_BLK // 2)
        updl = jnp.logical_and(is_left, lv < bvl)
        updr = jnp.logical_and(jnp.logical_not(is_left), lv < bvr)
        return (jnp.where(updl, lv, bvl), jnp.where(updl, li, bil),
                jnp.where(updr, lv, bvr), jnp.where(updr, li, bir))

    bv0 = jnp.full((TOK_BLK,), jnp.inf, dtype=jnp.float32)
    bi0 = jnp.zeros((TOK_BLK,), dtype=jnp.int32)
    bvl, bil, bvr, bir = lax.fori_loop(0, N_EMB // CODE_BLK, step,
                                       (bv0, bi0, bv0, bi0))
    bvl_q = bvl.astype(jnp.bfloat16).astype(jnp.float32)
    idx_ref[0, 0, :] = jnp.where(bvr < bvl_q, bir, bil)


_argmin_call = pl.pallas_call(
    _argmin_body,
    grid=(N_TOK_BLKS,),
    in_specs=[
        pl.BlockSpec((TOK_BLK, D), lambda i: (i, 0)),
        pl.BlockSpec((N_EMB, D), lambda i: (0, 0)),
        pl.BlockSpec((TOK_BLK, 1), lambda i: (i, 0)),
        pl.BlockSpec((1, N_EMB), lambda i: (0, 0)),
    ],
    out_specs=pl.BlockSpec((1, 1, TOK_BLK), lambda i: (i, 0, 0)),
    out_shape=jax.ShapeDtypeStruct((N_TOK_BLKS, 1, TOK_BLK), jnp.int32),
)


_DPAD = 128


@functools.cache
def _build_sc_gather():
    @functools.partial(
        pl.kernel,
        out_type=jax.ShapeDtypeStruct((N_TOK, _DPAD), jnp.float32),
        mesh=plsc.VectorSubcoreMesh(core_axis_name="c", subcore_axis_name="s"),
        scratch_types=[
            pltpu.VMEM((_NCHUNK, _CHUNK), jnp.int32),
            pltpu.VMEM((_BPW, _DPAD), jnp.float32),
            pltpu.SemaphoreType.DMA,
        ],
    )
    def _sc_gather(idx_hbm, emb_hbm, zq_hbm, idx_v, rows_v, sem):
        wid = lax.axis_index("s") * _NC + lax.axis_index("c")
        base = wid * _BPW
        pltpu.sync_copy(idx_hbm.at[wid], idx_v)
        copies = [
            pltpu.async_copy(emb_hbm.at[idx_v.at[j]],
                             rows_v.at[pl.ds(j * _CHUNK, _CHUNK)], sem)
            for j in range(_NCHUNK)
        ]
        for cp in copies:
            cp.wait()
        pltpu.sync_copy(rows_v, zq_hbm.at[pl.ds(base, _BPW)])

    return _sc_gather


def _finalize_body(ze_ref, zq_ref, idx_ref, st_ref, loss_ref, perp_ref,
                   sse_ref, ent_ref):
    i = pl.program_id(0)

    @pl.when(i == 0)
    def _():
        sse_ref[0] = 0.0
        ent_ref[0] = 0.0

    ze = ze_ref[...]
    zq = zq_ref[:, :D]
    diff = zq - ze
    st_ref[...] = ze + diff
    sse_ref[0] += jnp.sum(diff * diff)

    idx = idx_ref[...]
    jcol = i * CNT_BLK + lax.broadcasted_iota(jnp.int32, (1, CNT_BLK), 1)
    counts = jnp.sum((idx == jcol).astype(jnp.float32), axis=0)
    p = counts * (1.0 / N_TOK)
    ent_ref[0] += jnp.sum(p * jnp.log(p + 1e-10))

    @pl.when(i == N_TOK_BLKS - 1)
    def _():
        loss_ref[0, 0] = sse_ref[0] * (1.25 / (N_TOK * D))
        perp_ref[0, 0] = jnp.exp(-ent_ref[0])


_finalize_call = pl.pallas_call(
    _finalize_body,
    grid=(N_TOK_BLKS,),
    in_specs=[
        pl.BlockSpec((TOK_BLK, D), lambda i: (i, 0)),
        pl.BlockSpec((TOK_BLK, _DPAD), lambda i: (i, 0)),
        pl.BlockSpec((N_TOK, 1), lambda i: (0, 0)),
    ],
    out_specs=[
        pl.BlockSpec((TOK_BLK, D), lambda i: (i, 0)),
        pl.BlockSpec((1, 1), lambda i: (0, 0), memory_space=pltpu.SMEM),
        pl.BlockSpec((1, 1), lambda i: (0, 0), memory_space=pltpu.SMEM),
    ],
    out_shape=[
        jax.ShapeDtypeStruct((N_TOK, D), jnp.float32),
        jax.ShapeDtypeStruct((1, 1), jnp.float32),
        jax.ShapeDtypeStruct((1, 1), jnp.float32),
    ],
    scratch_shapes=[
        pltpu.SMEM((1,), jnp.float32),
        pltpu.SMEM((1,), jnp.float32),
    ],
)


def kernel(z_e, embedding):
    z_e_flat = z_e.reshape(N_TOK, D)
    z2 = jnp.sum(z_e_flat ** 2, axis=1, keepdims=True)
    e2 = jnp.sum(embedding ** 2, axis=1).reshape(1, N_EMB)
    idx3 = _argmin_call(z_e_flat, embedding, z2, e2)
    indices = idx3.reshape(N_TOK)
    emb_pad = jnp.pad(embedding, ((0, 0), (0, _DPAD - D)))
    zq_pad = _build_sc_gather()(indices.reshape(_NW, _NCHUNK, _CHUNK),
                                emb_pad)
    st, loss, perp = _finalize_call(z_e_flat, zq_pad,
                                    indices.reshape(N_TOK, 1))
    return st.reshape(z_e.shape), loss[0, 0], perp[0, 0]


# speedup vs baseline: 4.6678x; 1.1095x over previous
import functools

import jax
import jax.numpy as jnp
from jax import lax
from jax.experimental import pallas as pl
from jax.experimental.pallas import tpu as pltpu
from jax.experimental.pallas import tpu_sc as plsc

N_EMB = 8192
D = 32
N_TOK = 8192
TOK_BLK = 1024
CODE_BLK = 4096
N_TOK_BLKS = N_TOK // TOK_BLK
CNT_BLK = N_EMB // N_TOK_BLKS

_NC, _NS = 2, 16
_NW = _NC * _NS
_BPW = N_TOK // _NW
_CHUNK = 128
_NCHUNK = _BPW // _CHUNK


def _argmin_body(ze_ref, emb_ref, z2_ref, e2_ref, idx_ref):
    z = ze_ref[...]
    z2 = z2_ref[...]

    def step(c, carry):
        bvl, bil, bvr, bir = carry
        e = emb_ref[pl.ds(c * CODE_BLK, CODE_BLK), :]
        e2 = e2_ref[0, pl.ds(c * CODE_BLK, CODE_BLK)]
        mm = lax.dot_general(z, e, (((1,), (1,)), ((), ())),
                             preferred_element_type=jnp.float32)
        dist = (z2 + e2[None, :]) - 2.0 * mm
        lv = jnp.min(dist, axis=1)
        ii = lax.broadcasted_iota(jnp.int32, dist.shape, 1)
        li = jnp.min(jnp.where(dist == lv[:, None], ii, N_EMB), axis=1)
        li = li + c * CODE_BLK
        is_left = c < (N_EMB // CODE_BLK // 2)
        updl = jnp.logical_and(is_left, lv < bvl)
        updr = jnp.logical_and(jnp.logical_not(is_left), lv < bvr)
        return (jnp.where(updl, lv, bvl), jnp.where(updl, li, bil),
                jnp.where(updr, lv, bvr), jnp.where(updr, li, bir))

    bv0 = jnp.full((TOK_BLK,), jnp.inf, dtype=jnp.float32)
    bi0 = jnp.zeros((TOK_BLK,), dtype=jnp.int32)
    bvl, bil, bvr, bir = lax.fori_loop(0, N_EMB // CODE_BLK, step,
                                       (bv0, bi0, bv0, bi0))
    bvl_q = bvl.astype(jnp.bfloat16).astype(jnp.float32)
    idx_ref[0, 0, :] = jnp.where(bvr < bvl_q, bir, bil)


_argmin_call = pl.pallas_call(
    _argmin_body,
    grid=(N_TOK_BLKS,),
    in_specs=[
        pl.BlockSpec((TOK_BLK, D), lambda i: (i, 0)),
        pl.BlockSpec((N_EMB, D), lambda i: (0, 0)),
        pl.BlockSpec((TOK_BLK, 1), lambda i: (i, 0)),
        pl.BlockSpec((1, N_EMB), lambda i: (0, 0)),
    ],
    out_specs=pl.BlockSpec((1, 1, TOK_BLK), lambda i: (i, 0, 0)),
    out_shape=jax.ShapeDtypeStruct((N_TOK_BLKS, 1, TOK_BLK), jnp.int32),
)


_DPAD = 128


@functools.cache
def _build_sc_gather():
    @functools.partial(
        pl.kernel,
        out_type=jax.ShapeDtypeStruct((N_TOK, _DPAD), jnp.float32),
        mesh=plsc.VectorSubcoreMesh(core_axis_name="c", subcore_axis_name="s"),
        scratch_types=[
            pltpu.VMEM((_NCHUNK, _CHUNK), jnp.int32),
            pltpu.VMEM((_BPW, _DPAD), jnp.float32),
            pltpu.SemaphoreType.DMA,
        ],
    )
    def _sc_gather(idx_hbm, emb_hbm, zq_hbm, idx_v, rows_v, sem):
        wid = lax.axis_index("s") * _NC + lax.axis_index("c")
        base = wid * _BPW
        pltpu.sync_copy(idx_hbm.at[wid], idx_v)
        copies = [
            pltpu.async_copy(emb_hbm.at[idx_v.at[j]],
                             rows_v.at[pl.ds(j * _CHUNK, _CHUNK)], sem)
            for j in range(_NCHUNK)
        ]
        for cp in copies:
            cp.wait()
        pltpu.sync_copy(rows_v, zq_hbm.at[pl.ds(base, _BPW)])

    return _sc_gather


def _finalize_body(ze_ref, zq_ref, idx_ref, st_ref, loss_ref, perp_ref,
                   sse_ref, ent_ref):
    i = pl.program_id(0)

    @pl.when(i == 0)
    def _():
        sse_ref[0] = 0.0
        ent_ref[0] = 0.0

    ze = ze_ref[...]
    zq = zq_ref[:, :D]
    diff = zq - ze
    st_ref[...] = ze + diff
    sse_ref[0] += jnp.sum(diff * diff)

    idx = idx_ref[...]
    jcol = i * CNT_BLK + lax.broadcasted_iota(jnp.int32, (1, CNT_BLK), 1)
    counts = jnp.sum((idx == jcol).astype(jnp.float32), axis=0)
    p = counts * (1.0 / N_TOK)
    ent_ref[0] += jnp.sum(p * jnp.log(p + 1e-10))

    @pl.when(i == N_TOK_BLKS - 1)
    def _():
        loss_ref[0, 0] = sse_ref[0] * (1.25 / (N_TOK * D))
        perp_ref[0, 0] = jnp.exp(-ent_ref[0])


_finalize_call = pl.pallas_call(
    _finalize_body,
    grid=(N_TOK_BLKS,),
    in_specs=[
        pl.BlockSpec((TOK_BLK, D), lambda i: (i, 0)),
        pl.BlockSpec((TOK_BLK, _DPAD), lambda i: (i, 0)),
        pl.BlockSpec((N_TOK, 1), lambda i: (0, 0)),
    ],
    out_specs=[
        pl.BlockSpec((TOK_BLK, D), lambda i: (i, 0)),
        pl.BlockSpec((1, 1), lambda i: (0, 0), memory_space=pltpu.SMEM),
        pl.BlockSpec((1, 1), lambda i: (0, 0), memory_space=pltpu.SMEM),
    ],
    out_shape=[
        jax.ShapeDtypeStruct((N_TOK, D), jnp.float32),
        jax.ShapeDtypeStruct((1, 1), jnp.float32),
        jax.ShapeDtypeStruct((1, 1), jnp.float32),
    ],
    scratch_shapes=[
        pltpu.SMEM((1,), jnp.float32),
        pltpu.SMEM((1,), jnp.float32),
    ],
)


def kernel(z_e, embedding):
    z_e_flat = z_e.reshape(N_TOK, D)
    z2 = jnp.sum(z_e_flat ** 2, axis=1, keepdims=True)
    e2 = jnp.sum(embedding ** 2, axis=1).reshape(1, N_EMB)
    idx3 = _argmin_call(z_e_flat, embedding, z2, e2)
    indices = idx3.reshape(N_TOK)
    emb_pad = jnp.pad(embedding, ((0, 0), (0, _DPAD - D)))
    zq_pad = _build_sc_gather()(indices.reshape(_NW, _NCHUNK, _CHUNK),
                                emb_pad)
    st, loss, perp = _finalize_call(z_e_flat, zq_pad,
                                    indices.reshape(N_TOK, 1))
    return st.reshape(z_e.shape), loss[0, 0], perp[0, 0]


# speedup vs baseline: 4.7496x; 1.0175x over previous
import functools

import jax
import jax.numpy as jnp
from jax import lax
from jax.experimental import pallas as pl
from jax.experimental.pallas import tpu as pltpu
from jax.experimental.pallas import tpu_sc as plsc

N_EMB = 8192
D = 32
N_TOK = 8192
TOK_BLK = 1024
CODE_BLK = 4096
N_TOK_BLKS = N_TOK // TOK_BLK
CNT_BLK = N_EMB // N_TOK_BLKS

_NC, _NS = 2, 16
_NW = _NC * _NS
_BPW = N_TOK // _NW
_CHUNK = 128
_NCHUNK = _BPW // _CHUNK


def _argmin_body(ze_ref, emb_ref, z2_ref, e2_ref, idx_ref):
    z = ze_ref[...]
    z2 = z2_ref[...]

    def half_argmin(h):
        e = emb_ref[pl.ds(h * CODE_BLK, CODE_BLK), :]
        e2 = e2_ref[0, pl.ds(h * CODE_BLK, CODE_BLK)]
        mm = lax.dot_general(z, e, (((1,), (1,)), ((), ())),
                             preferred_element_type=jnp.float32)
        dist = (z2 + e2[None, :]) - 2.0 * mm
        lv = jnp.min(dist, axis=1)
        ii = lax.broadcasted_iota(jnp.int32, dist.shape, 1)
        li = jnp.min(jnp.where(dist == lv[:, None], ii, N_EMB), axis=1)
        return lv, li + h * CODE_BLK

    bvl, bil = half_argmin(0)
    bvr, bir = half_argmin(1)
    bvl_q = bvl.astype(jnp.bfloat16).astype(jnp.float32)
    idx_ref[0, 0, :] = jnp.where(bvr < bvl_q, bir, bil)


_argmin_call = pl.pallas_call(
    _argmin_body,
    grid=(N_TOK_BLKS,),
    in_specs=[
        pl.BlockSpec((TOK_BLK, D), lambda i: (i, 0)),
        pl.BlockSpec((N_EMB, D), lambda i: (0, 0)),
        pl.BlockSpec((TOK_BLK, 1), lambda i: (i, 0)),
        pl.BlockSpec((1, N_EMB), lambda i: (0, 0)),
    ],
    out_specs=pl.BlockSpec((1, 1, TOK_BLK), lambda i: (i, 0, 0)),
    out_shape=jax.ShapeDtypeStruct((N_TOK_BLKS, 1, TOK_BLK), jnp.int32),
)


_DPAD = 128


@functools.cache
def _build_sc_gather():
    @functools.partial(
        pl.kernel,
        out_type=jax.ShapeDtypeStruct((N_TOK, _DPAD), jnp.float32),
        mesh=plsc.VectorSubcoreMesh(core_axis_name="c", subcore_axis_name="s"),
        scratch_types=[
            pltpu.VMEM((_NCHUNK, _CHUNK), jnp.int32),
            pltpu.VMEM((_BPW, _DPAD), jnp.float32),
            pltpu.SemaphoreType.DMA,
        ],
    )
    def _sc_gather(idx_hbm, emb_hbm, zq_hbm, idx_v, rows_v, sem):
        wid = lax.axis_index("s") * _NC + lax.axis_index("c")
        base = wid * _BPW
        pltpu.sync_copy(idx_hbm.at[wid], idx_v)
        copies = [
            pltpu.async_copy(emb_hbm.at[idx_v.at[j]],
                             rows_v.at[pl.ds(j * _CHUNK, _CHUNK)], sem)
            for j in range(_NCHUNK)
        ]
        for cp in copies:
            cp.wait()
        pltpu.sync_copy(rows_v, zq_hbm.at[pl.ds(base, _BPW)])

    return _sc_gather


def _finalize_body(ze_ref, zq_ref, idx_ref, st_ref, loss_ref, perp_ref,
                   sse_ref, ent_ref):
    i = pl.program_id(0)

    @pl.when(i == 0)
    def _():
        sse_ref[0] = 0.0
        ent_ref[0] = 0.0

    ze = ze_ref[...]
    zq = zq_ref[:, :D]
    diff = zq - ze
    st_ref[...] = ze + diff
    sse_ref[0] += jnp.sum(diff * diff)

    idx = idx_ref[...]
    jcol = i * CNT_BLK + lax.broadcasted_iota(jnp.int32, (1, CNT_BLK), 1)
    counts = jnp.sum((idx == jcol).astype(jnp.float32), axis=0)
    p = counts * (1.0 / N_TOK)
    ent_ref[0] += jnp.sum(p * jnp.log(p + 1e-10))

    @pl.when(i == N_TOK_BLKS - 1)
    def _():
        loss_ref[0, 0] = sse_ref[0] * (1.25 / (N_TOK * D))
        perp_ref[0, 0] = jnp.exp(-ent_ref[0])


_finalize_call = pl.pallas_call(
    _finalize_body,
    grid=(N_TOK_BLKS,),
    in_specs=[
        pl.BlockSpec((TOK_BLK, D), lambda i: (i, 0)),
        pl.BlockSpec((TOK_BLK, _DPAD), lambda i: (i, 0)),
        pl.BlockSpec((N_TOK, 1), lambda i: (0, 0)),
    ],
    out_specs=[
        pl.BlockSpec((TOK_BLK, D), lambda i: (i, 0)),
        pl.BlockSpec((1, 1), lambda i: (0, 0), memory_space=pltpu.SMEM),
        pl.BlockSpec((1, 1), lambda i: (0, 0), memory_space=pltpu.SMEM),
    ],
    out_shape=[
        jax.ShapeDtypeStruct((N_TOK, D), jnp.float32),
        jax.ShapeDtypeStruct((1, 1), jnp.float32),
        jax.ShapeDtypeStruct((1, 1), jnp.float32),
    ],
    scratch_shapes=[
        pltpu.SMEM((1,), jnp.float32),
        pltpu.SMEM((1,), jnp.float32),
    ],
)


def kernel(z_e, embedding):
    z_e_flat = z_e.reshape(N_TOK, D)
    z2 = jnp.sum(z_e_flat ** 2, axis=1, keepdims=True)
    e2 = jnp.sum(embedding ** 2, axis=1).reshape(1, N_EMB)
    idx3 = _argmin_call(z_e_flat, embedding, z2, e2)
    indices = idx3.reshape(N_TOK)
    emb_pad = jnp.pad(embedding, ((0, 0), (0, _DPAD - D)))
    zq_pad = _build_sc_gather()(indices.reshape(_NW, _NCHUNK, _CHUNK),
                                emb_pad)
    st, loss, perp = _finalize_call(z_e_flat, zq_pad,
                                    indices.reshape(N_TOK, 1))
    return st.reshape(z_e.shape), loss[0, 0], perp[0, 0]


# speedup vs baseline: 5.5951x; 1.1780x over previous
import functools

import jax
import jax.numpy as jnp
from jax import lax
from jax.experimental import pallas as pl
from jax.experimental.pallas import tpu as pltpu
from jax.experimental.pallas import tpu_sc as plsc

N_EMB = 8192
D = 32
N_TOK = 8192
TOK_BLK = 1024
CODE_BLK = 4096
N_TOK_BLKS = N_TOK // TOK_BLK
CNT_BLK = N_EMB // N_TOK_BLKS

_NC, _NS = 2, 16
_NW = _NC * _NS
_BPW = N_TOK // _NW
_CHUNK = 128
_NCHUNK = _BPW // _CHUNK


def _argmin_body(ze_ref, emb_ref, z2_ref, e2_ref, idx_ref):
    z = ze_ref[...]
    z2 = z2_ref[...]

    def half_argmin(h):
        e = emb_ref[pl.ds(h * CODE_BLK, CODE_BLK), :]
        e2 = e2_ref[0, pl.ds(h * CODE_BLK, CODE_BLK)]
        mm = lax.dot_general(z, e, (((1,), (1,)), ((), ())),
                             preferred_element_type=jnp.float32)
        dist = (z2 + e2[None, :]) - 2.0 * mm
        lv = jnp.min(dist, axis=1)
        ii = lax.broadcasted_iota(jnp.int32, dist.shape, 1)
        li = jnp.min(jnp.where(dist == lv[:, None], ii, N_EMB), axis=1)
        return lv, li + h * CODE_BLK

    bvl, bil = half_argmin(0)
    bvr, bir = half_argmin(1)
    bvl_q = bvl.astype(jnp.bfloat16).astype(jnp.float32)
    idx_ref[0, 0, :] = jnp.where(bvr < bvl_q, bir, bil)


_argmin_call = pl.pallas_call(
    _argmin_body,
    grid=(N_TOK_BLKS,),
    in_specs=[
        pl.BlockSpec((TOK_BLK, D), lambda i: (i, 0)),
        pl.BlockSpec((N_EMB, D), lambda i: (0, 0)),
        pl.BlockSpec((TOK_BLK, 1), lambda i: (i, 0)),
        pl.BlockSpec((1, N_EMB), lambda i: (0, 0)),
    ],
    out_specs=pl.BlockSpec((1, 1, TOK_BLK), lambda i: (i, 0, 0)),
    out_shape=jax.ShapeDtypeStruct((N_TOK_BLKS, 1, TOK_BLK), jnp.int32),
)


_DPAD = 128


@functools.cache
def _build_sc_gather():
    @functools.partial(
        pl.kernel,
        out_type=(jax.ShapeDtypeStruct((N_TOK, _DPAD), jnp.float32),
                  jax.ShapeDtypeStruct((_NC, N_EMB), jnp.float32)),
        mesh=plsc.VectorSubcoreMesh(core_axis_name="c", subcore_axis_name="s"),
        scratch_types=[
            pltpu.VMEM((_NCHUNK, _CHUNK), jnp.int32),
            pltpu.VMEM((_BPW, _DPAD), jnp.float32),
            pltpu.VMEM((_CHUNK,), jnp.float32),
            pltpu.VMEM((N_EMB // _NS,), jnp.float32),
            pltpu.VMEM_SHARED((N_EMB,), jnp.float32),
            pltpu.SemaphoreType.DMA,
        ],
    )
    def _sc_gather(idx_hbm, emb_hbm, zq_hbm, cnt_hbm, idx_v, rows_v,
                   ones_v, zero_v, cnt_sh, sem):
        cid = lax.axis_index("c")
        sid = lax.axis_index("s")
        wid = sid * _NC + cid
        base = wid * _BPW
        pltpu.sync_copy(idx_hbm.at[wid], idx_v)
        copies = [
            pltpu.async_copy(emb_hbm.at[idx_v.at[j]],
                             rows_v.at[pl.ds(j * _CHUNK, _CHUNK)], sem)
            for j in range(_NCHUNK)
        ]
        stripe = N_EMB // _NS
        for k in range(stripe // 16):
            zero_v[pl.ds(k * 16, 16)] = jnp.zeros((16,), jnp.float32)
        for k in range(_CHUNK // 16):
            ones_v[pl.ds(k * 16, 16)] = jnp.full((16,), 1.0, jnp.float32)
        pltpu.sync_copy(zero_v, cnt_sh.at[pl.ds(sid * stripe, stripe)])
        plsc.subcore_barrier()
        for j in range(_NCHUNK):
            pltpu.sync_copy(ones_v, cnt_sh.at[idx_v.at[j]], add=True)
        plsc.subcore_barrier()

        @pl.when(sid == 0)
        def _():
            pltpu.sync_copy(cnt_sh, cnt_hbm.at[cid])

        for cp in copies:
            cp.wait()
        pltpu.sync_copy(rows_v, zq_hbm.at[pl.ds(base, _BPW)])

    return _sc_gather


def _finalize_body(ze_ref, zq_ref, cnt_ref, st_ref, loss_ref, perp_ref,
                   sse_ref):
    i = pl.program_id(0)

    @pl.when(i == 0)
    def _():
        sse_ref[0] = 0.0

    ze = ze_ref[...]
    zq = zq_ref[:, :D]
    diff = zq - ze
    st_ref[...] = ze + diff
    sse_ref[0] += jnp.sum(diff * diff)

    @pl.when(i == N_TOK_BLKS - 1)
    def _():
        counts = cnt_ref[0, :] + cnt_ref[1, :]
        p = counts * (1.0 / N_TOK)
        ent = jnp.sum(p * jnp.log(p + 1e-10))
        loss_ref[0, 0] = sse_ref[0] * (1.25 / (N_TOK * D))
        perp_ref[0, 0] = jnp.exp(-ent)


_finalize_call = pl.pallas_call(
    _finalize_body,
    grid=(N_TOK_BLKS,),
    in_specs=[
        pl.BlockSpec((TOK_BLK, D), lambda i: (i, 0)),
        pl.BlockSpec((TOK_BLK, _DPAD), lambda i: (i, 0)),
        pl.BlockSpec((_NC, N_EMB), lambda i: (0, 0)),
    ],
    out_specs=[
        pl.BlockSpec((TOK_BLK, D), lambda i: (i, 0)),
        pl.BlockSpec((1, 1), lambda i: (0, 0), memory_space=pltpu.SMEM),
        pl.BlockSpec((1, 1), lambda i: (0, 0), memory_space=pltpu.SMEM),
    ],
    out_shape=[
        jax.ShapeDtypeStruct((N_TOK, D), jnp.float32),
        jax.ShapeDtypeStruct((1, 1), jnp.float32),
        jax.ShapeDtypeStruct((1, 1), jnp.float32),
    ],
    scratch_shapes=[
        pltpu.SMEM((1,), jnp.float32),
    ],
)


def kernel(z_e, embedding):
    z_e_flat = z_e.reshape(N_TOK, D)
    z2 = jnp.sum(z_e_flat ** 2, axis=1, keepdims=True)
    e2 = jnp.sum(embedding ** 2, axis=1).reshape(1, N_EMB)
    idx3 = _argmin_call(z_e_flat, embedding, z2, e2)
    indices = idx3.reshape(N_TOK)
    emb_pad = jnp.pad(embedding, ((0, 0), (0, _DPAD - D)))
    zq_pad, counts = _build_sc_gather()(indices.reshape(_NW, _NCHUNK, _CHUNK),
                                        emb_pad)
    st, loss, perp = _finalize_call(z_e_flat, zq_pad, counts)
    return st.reshape(z_e.shape), loss[0, 0], perp[0, 0]


# speedup vs baseline: 5.5975x; 1.0004x over previous
import functools

import jax
import jax.numpy as jnp
from jax import lax
from jax.experimental import pallas as pl
from jax.experimental.pallas import tpu as pltpu
from jax.experimental.pallas import tpu_sc as plsc

N_EMB = 8192
D = 32
N_TOK = 8192
TOK_BLK = 1024
CODE_BLK = 4096
N_TOK_BLKS = N_TOK // TOK_BLK
CNT_BLK = N_EMB // N_TOK_BLKS

_NC, _NS = 2, 16
_NW = _NC * _NS
_BPW = N_TOK // _NW
_CHUNK = 128
_NCHUNK = _BPW // _CHUNK


def _argmin_body(ze_ref, emb_ref, z2_ref, e2_ref, idx_ref):
    z = ze_ref[...]
    z2 = z2_ref[...]

    mm_full = lax.dot_general(z, emb_ref[...], (((1,), (1,)), ((), ())),
                              preferred_element_type=jnp.float32)

    def half_argmin(h):
        e2 = e2_ref[0, pl.ds(h * CODE_BLK, CODE_BLK)]
        mm = mm_full[:, h * CODE_BLK:(h + 1) * CODE_BLK]
        dist = (z2 + e2[None, :]) - 2.0 * mm
        lv = jnp.min(dist, axis=1)
        ii = lax.broadcasted_iota(jnp.int32, dist.shape, 1)
        li = jnp.min(jnp.where(dist == lv[:, None], ii, N_EMB), axis=1)
        return lv, li + h * CODE_BLK

    bvl, bil = half_argmin(0)
    bvr, bir = half_argmin(1)
    bvl_q = bvl.astype(jnp.bfloat16).astype(jnp.float32)
    idx_ref[0, 0, :] = jnp.where(bvr < bvl_q, bir, bil)


_argmin_call = pl.pallas_call(
    _argmin_body,
    grid=(N_TOK_BLKS,),
    in_specs=[
        pl.BlockSpec((TOK_BLK, D), lambda i: (i, 0)),
        pl.BlockSpec((N_EMB, D), lambda i: (0, 0)),
        pl.BlockSpec((TOK_BLK, 1), lambda i: (i, 0)),
        pl.BlockSpec((1, N_EMB), lambda i: (0, 0)),
    ],
    out_specs=pl.BlockSpec((1, 1, TOK_BLK), lambda i: (i, 0, 0)),
    out_shape=jax.ShapeDtypeStruct((N_TOK_BLKS, 1, TOK_BLK), jnp.int32),
)


_DPAD = 128


@functools.cache
def _build_sc_gather():
    @functools.partial(
        pl.kernel,
        out_type=(jax.ShapeDtypeStruct((N_TOK, _DPAD), jnp.float32),
                  jax.ShapeDtypeStruct((_NC, N_EMB), jnp.float32)),
        mesh=plsc.VectorSubcoreMesh(core_axis_name="c", subcore_axis_name="s"),
        scratch_types=[
            pltpu.VMEM((_NCHUNK, _CHUNK), jnp.int32),
            pltpu.VMEM((_BPW, _DPAD), jnp.float32),
            pltpu.VMEM((_CHUNK,), jnp.float32),
            pltpu.VMEM((N_EMB // _NS,), jnp.float32),
            pltpu.VMEM_SHARED((N_EMB,), jnp.float32),
            pltpu.SemaphoreType.DMA,
        ],
    )
    def _sc_gather(idx_hbm, emb_hbm, zq_hbm, cnt_hbm, idx_v, rows_v,
                   ones_v, zero_v, cnt_sh, sem):
        cid = lax.axis_index("c")
        sid = lax.axis_index("s")
        wid = sid * _NC + cid
        base = wid * _BPW
        pltpu.sync_copy(idx_hbm.at[wid], idx_v)
        copies = [
            pltpu.async_copy(emb_hbm.at[idx_v.at[j]],
                             rows_v.at[pl.ds(j * _CHUNK, _CHUNK)], sem)
            for j in range(_NCHUNK)
        ]
        stripe = N_EMB // _NS
        for k in range(stripe // 16):
            zero_v[pl.ds(k * 16, 16)] = jnp.zeros((16,), jnp.float32)
        for k in range(_CHUNK // 16):
            ones_v[pl.ds(k * 16, 16)] = jnp.full((16,), 1.0, jnp.float32)
        pltpu.sync_copy(zero_v, cnt_sh.at[pl.ds(sid * stripe, stripe)])
        plsc.subcore_barrier()
        for j in range(_NCHUNK):
            pltpu.sync_copy(ones_v, cnt_sh.at[idx_v.at[j]], add=True)
        plsc.subcore_barrier()

        @pl.when(sid == 0)
        def _():
            pltpu.sync_copy(cnt_sh, cnt_hbm.at[cid])

        for cp in copies:
            cp.wait()
        pltpu.sync_copy(rows_v, zq_hbm.at[pl.ds(base, _BPW)])

    return _sc_gather


def _finalize_body(ze_ref, zq_ref, cnt_ref, st_ref, loss_ref, perp_ref,
                   sse_ref):
    i = pl.program_id(0)

    @pl.when(i == 0)
    def _():
        sse_ref[0] = 0.0

    ze = ze_ref[...]
    zq = zq_ref[:, :D]
    diff = zq - ze
    st_ref[...] = ze + diff
    sse_ref[0] += jnp.sum(diff * diff)

    @pl.when(i == N_TOK_BLKS - 1)
    def _():
        counts = cnt_ref[0, :] + cnt_ref[1, :]
        p = counts * (1.0 / N_TOK)
        ent = jnp.sum(p * jnp.log(p + 1e-10))
        loss_ref[0, 0] = sse_ref[0] * (1.25 / (N_TOK * D))
        perp_ref[0, 0] = jnp.exp(-ent)


_finalize_call = pl.pallas_call(
    _finalize_body,
    grid=(N_TOK_BLKS,),
    in_specs=[
        pl.BlockSpec((TOK_BLK, D), lambda i: (i, 0)),
        pl.BlockSpec((TOK_BLK, _DPAD), lambda i: (i, 0)),
        pl.BlockSpec((_NC, N_EMB), lambda i: (0, 0)),
    ],
    out_specs=[
        pl.BlockSpec((TOK_BLK, D), lambda i: (i, 0)),
        pl.BlockSpec((1, 1), lambda i: (0, 0), memory_space=pltpu.SMEM),
        pl.BlockSpec((1, 1), lambda i: (0, 0), memory_space=pltpu.SMEM),
    ],
    out_shape=[
        jax.ShapeDtypeStruct((N_TOK, D), jnp.float32),
        jax.ShapeDtypeStruct((1, 1), jnp.float32),
        jax.ShapeDtypeStruct((1, 1), jnp.float32),
    ],
    scratch_shapes=[
        pltpu.SMEM((1,), jnp.float32),
    ],
)


def kernel(z_e, embedding):
    z_e_flat = z_e.reshape(N_TOK, D)
    z2 = jnp.sum(z_e_flat ** 2, axis=1, keepdims=True)
    e2 = jnp.sum(embedding ** 2, axis=1).reshape(1, N_EMB)
    idx3 = _argmin_call(z_e_flat, embedding, z2, e2)
    indices = idx3.reshape(N_TOK)
    emb_pad = jnp.pad(embedding, ((0, 0), (0, _DPAD - D)))
    zq_pad, counts = _build_sc_gather()(indices.reshape(_NW, _NCHUNK, _CHUNK),
                                        emb_pad)
    st, loss, perp = _finalize_call(z_e_flat, zq_pad, counts)
    return st.reshape(z_e.shape), loss[0, 0], perp[0, 0]


# speedup vs baseline: 5.5993x; 1.0003x over previous
import functools

import jax
import jax.numpy as jnp
from jax import lax
from jax.experimental import pallas as pl
from jax.experimental.pallas import tpu as pltpu
from jax.experimental.pallas import tpu_sc as plsc

N_EMB = 8192
D = 32
N_TOK = 8192
TOK_BLK = 1024
CODE_BLK = 4096
N_TOK_BLKS = N_TOK // TOK_BLK

_NC, _NS = 2, 16
_NW = _NC * _NS
_BPW = N_TOK // _NW
_CHUNK = 128
_NCHUNK = _BPW // _CHUNK


def _argmin_body(ze_ref, emb_ref, z2_ref, e2_ref, idx_ref):
    z = ze_ref[...]
    z2 = z2_ref[...]

    mm_full = lax.dot_general(z, emb_ref[...], (((1,), (1,)), ((), ())),
                              preferred_element_type=jnp.float32)

    def half_argmin(h):
        e2 = e2_ref[0, pl.ds(h * CODE_BLK, CODE_BLK)]
        mm = mm_full[:, h * CODE_BLK:(h + 1) * CODE_BLK]
        dist = (z2 + e2[None, :]) - 2.0 * mm
        lv = jnp.min(dist, axis=1)
        ii = lax.broadcasted_iota(jnp.int32, dist.shape, 1)
        li = jnp.min(jnp.where(dist == lv[:, None], ii, N_EMB), axis=1)
        return lv, li + h * CODE_BLK

    bvl, bil = half_argmin(0)
    bvr, bir = half_argmin(1)
    bvl_q = bvl.astype(jnp.bfloat16).astype(jnp.float32)
    idx_ref[0, 0, :] = jnp.where(bvr < bvl_q, bir, bil)


_argmin_call = pl.pallas_call(
    _argmin_body,
    grid=(N_TOK_BLKS,),
    in_specs=[
        pl.BlockSpec((TOK_BLK, D), lambda i: (i, 0)),
        pl.BlockSpec((N_EMB, D), lambda i: (0, 0)),
        pl.BlockSpec((TOK_BLK, 1), lambda i: (i, 0)),
        pl.BlockSpec((1, N_EMB), lambda i: (0, 0)),
    ],
    out_specs=pl.BlockSpec((1, 1, TOK_BLK), lambda i: (i, 0, 0)),
    out_shape=jax.ShapeDtypeStruct((N_TOK_BLKS, 1, TOK_BLK), jnp.int32),
)


_DPAD = 128


@functools.cache
def _build_sc_gather():
    @functools.partial(
        pl.kernel,
        out_type=(jax.ShapeDtypeStruct((N_TOK, _DPAD), jnp.float32),
                  jax.ShapeDtypeStruct((_NC, N_EMB), jnp.float32)),
        mesh=plsc.VectorSubcoreMesh(core_axis_name="c", subcore_axis_name="s"),
        scratch_types=[
            pltpu.VMEM((_NCHUNK, _CHUNK), jnp.int32),
            pltpu.VMEM((_BPW, _DPAD), jnp.float32),
            pltpu.VMEM((_CHUNK,), jnp.float32),
            pltpu.VMEM((N_EMB // _NS,), jnp.float32),
            pltpu.VMEM_SHARED((N_EMB,), jnp.float32),
            pltpu.SemaphoreType.DMA,
        ],
    )
    def _sc_gather(idx_hbm, emb_hbm, zq_hbm, cnt_hbm, idx_v, rows_v,
                   ones_v, zero_v, cnt_sh, sem):
        cid = lax.axis_index("c")
        sid = lax.axis_index("s")
        wid = sid * _NC + cid
        base = wid * _BPW
        pltpu.sync_copy(idx_hbm.at[wid], idx_v)
        copies = [
            pltpu.async_copy(emb_hbm.at[idx_v.at[j]],
                             rows_v.at[pl.ds(j * _CHUNK, _CHUNK)], sem)
            for j in range(_NCHUNK)
        ]
        stripe = N_EMB // _NS
        for k in range(stripe // 16):
            zero_v[pl.ds(k * 16, 16)] = jnp.zeros((16,), jnp.float32)
        for k in range(_CHUNK // 16):
            ones_v[pl.ds(k * 16, 16)] = jnp.full((16,), 1.0, jnp.float32)
        pltpu.sync_copy(zero_v, cnt_sh.at[pl.ds(sid * stripe, stripe)])
        plsc.subcore_barrier()
        for j in range(_NCHUNK):
            pltpu.sync_copy(ones_v, cnt_sh.at[idx_v.at[j]], add=True)
        plsc.subcore_barrier()

        @pl.when(sid == 0)
        def _():
            pltpu.sync_copy(cnt_sh, cnt_hbm.at[cid])

        for cp in copies:
            cp.wait()
        pltpu.sync_copy(rows_v, zq_hbm.at[pl.ds(base, _BPW)])

    return _sc_gather


def _finalize_body(ze_ref, zq_ref, cnt_ref, st_ref, loss_ref, perp_ref,
                   sse_ref):
    i = pl.program_id(0)

    @pl.when(i == 0)
    def _():
        sse_ref[0] = 0.0

    ze = ze_ref[...]
    zq = zq_ref[:, :D]
    diff = zq - ze
    st_ref[...] = ze + diff
    sse_ref[0] += jnp.sum(diff * diff)

    @pl.when(i == N_TOK_BLKS - 1)
    def _():
        counts = cnt_ref[0, :] + cnt_ref[1, :]
        p = counts * (1.0 / N_TOK)
        ent = jnp.sum(p * jnp.log(p + 1e-10))
        loss_ref[0, 0] = sse_ref[0] * (1.25 / (N_TOK * D))
        perp_ref[0, 0] = jnp.exp(-ent)


_finalize_call = pl.pallas_call(
    _finalize_body,
    grid=(N_TOK_BLKS,),
    in_specs=[
        pl.BlockSpec((TOK_BLK, D), lambda i: (i, 0)),
        pl.BlockSpec((TOK_BLK, _DPAD), lambda i: (i, 0)),
        pl.BlockSpec((_NC, N_EMB), lambda i: (0, 0)),
    ],
    out_specs=[
        pl.BlockSpec((TOK_BLK, D), lambda i: (i, 0)),
        pl.BlockSpec((1, 1), lambda i: (0, 0), memory_space=pltpu.SMEM),
        pl.BlockSpec((1, 1), lambda i: (0, 0), memory_space=pltpu.SMEM),
    ],
    out_shape=[
        jax.ShapeDtypeStruct((N_TOK, D), jnp.float32),
        jax.ShapeDtypeStruct((1, 1), jnp.float32),
        jax.ShapeDtypeStruct((1, 1), jnp.float32),
    ],
    scratch_shapes=[
        pltpu.SMEM((1,), jnp.float32),
    ],
)


def kernel(z_e, embedding):
    z_e_flat = z_e.reshape(N_TOK, D)
    z2 = jnp.sum(z_e_flat ** 2, axis=1, keepdims=True)
    e2 = jnp.sum(embedding ** 2, axis=1).reshape(1, N_EMB)
    idx3 = _argmin_call(z_e_flat, embedding, z2, e2)
    indices = idx3.reshape(N_TOK)
    emb_pad = jnp.pad(embedding, ((0, 0), (0, _DPAD - D)))
    zq_pad, counts = _build_sc_gather()(indices.reshape(_NW, _NCHUNK, _CHUNK),
                                        emb_pad)
    st, loss, perp = _finalize_call(z_e_flat, zq_pad, counts)
    return st.reshape(z_e.shape), loss[0, 0], perp[0, 0]
